# baseline (speedup 1.0000x reference)
import functools

import jax
import jax.numpy as jnp
from jax import lax
from jax.experimental import pallas as pl
from jax.experimental.pallas import tpu as pltpu
from jax.experimental.pallas import tpu_sc as plsc

VOCAB = 1000000
EMBED_DIM = 32
BATCH = 4096
SEQ_LEN = 200

NUM_CORES = 2
NUM_SUBCORES = 16
NW = NUM_CORES * NUM_SUBCORES
CHUNK = 128
NCH = SEQ_LEN
NBUF = 5
LOOKAHEAD = 3
GROUPS = CHUNK // 16


_MESH = plsc.VectorSubcoreMesh(
    core_axis_name="c", subcore_axis_name="s",
    num_cores=NUM_CORES, num_subcores=NUM_SUBCORES,
)

SLABW = 512
SLABG = SLABW // 4
NSLAB = VOCAB // SLABW
KRING = 3


@functools.partial(
    pl.kernel,
    out_type=jax.ShapeDtypeStruct((VOCAB // 4, 128), jnp.float32),
    mesh=_MESH,
    scratch_types=[
        pltpu.VMEM((KRING, 32, SLABW), jnp.float32),
        pltpu.VMEM((KRING, SLABG, 128), jnp.float32),
        pltpu.SemaphoreType.DMA((KRING,)),
        pltpu.SemaphoreType.DMA((KRING,)),
    ],
    compiler_params=pltpu.CompilerParams(
        use_tc_tiling_on_sc=True, needs_layout_passes=False,
        disable_bounds_checks=True),
)
def _sc_regroup(tt_hbm, tail_hbm, out_hbm, slab_v, lin_v, lsem, ssem):
    wid = lax.axis_index("s") * NUM_CORES + lax.axis_index("c")
    nk = jnp.where(wid < NSLAB - (NSLAB // NW) * NW, NSLAB // NW + 1,
                   NSLAB // NW)

    @pl.when(wid == NW - 1)
    def _():
        pltpu.sync_copy(tail_hbm, out_hbm.at[pl.ds(NSLAB * SLABG, 16)])
    lane = lax.iota(jnp.int32, 16)
    fvec = [lane, lane + 16]

    def load(k, slot):
        v0 = (wid + NW * k) * SLABW
        pltpu.async_copy(tt_hbm.at[:, pl.ds(v0, SLABW)], slab_v.at[slot],
                         lsem.at[slot])

    @pl.when(nk > 0)
    def _():
        load(0, 0)

    @pl.when(nk > 1)
    def _():
        load(1, 1)

    @pl.loop(0, nk)
    def _(k):
        slot = lax.rem(k, KRING)
        v0 = (wid + NW * k) * SLABW
        pltpu.make_async_copy(
            tt_hbm.at[:, pl.ds(v0, SLABW)], slab_v.at[slot],
            lsem.at[slot]).wait()

        kn = k + 2
        sn = lax.rem(kn, KRING)

        @pl.when(kn < nk)
        def _():
            load(kn, sn)

        @pl.when(k >= KRING)
        def _():
            kp = k - KRING
            g0p = (wid + NW * kp) * SLABG
            pltpu.make_async_copy(
                lin_v.at[slot], out_hbm.at[pl.ds(g0p, SLABG)],
                ssem.at[slot]).wait()

        @pl.loop(0, SLABG, unroll=4)
        def _(r):
            col = [jnp.full((16,), 4 * r + a, jnp.int32) for a in range(4)]
            for h in range(8):
                x = plsc.load_gather(slab_v.at[slot], [fvec[h % 2],
                                                       col[h // 2]])
                lin_v[slot, r, pl.ds(16 * h, 16)] = x

        g0 = (wid + NW * k) * SLABG
        pltpu.async_copy(lin_v.at[slot], out_hbm.at[pl.ds(g0, SLABG)],
                         ssem.at[slot])

    @pl.loop(lax.max(nk - KRING, 0), nk)
    def _(k):
        slot = lax.rem(k, KRING)
        g0 = (wid + NW * k) * SLABG
        pltpu.make_async_copy(
            lin_v.at[slot], out_hbm.at[pl.ds(g0, SLABG)],
            ssem.at[slot]).wait()


@functools.partial(
    pl.kernel,
    out_type=jax.ShapeDtypeStruct((SEQ_LEN, EMBED_DIM, BATCH), jnp.float32),
    mesh=_MESH,
    scratch_types=[
        pltpu.VMEM((SEQ_LEN, CHUNK), jnp.int32),
        pltpu.VMEM((NBUF, CHUNK), jnp.int32),
        pltpu.VMEM((NBUF, CHUNK, 128), jnp.float32),
        pltpu.VMEM((NBUF, EMBED_DIM, CHUNK), jnp.float32),
        pltpu.SemaphoreType.DMA((NBUF,)),
        pltpu.SemaphoreType.DMA((NBUF,)),
    ],
    compiler_params=pltpu.CompilerParams(
        use_tc_tiling_on_sc=True, needs_layout_passes=False,
        disable_bounds_checks=True),
)
def _sc_gather(data_hbm, tbl_hbm, out_hbm, idx_v, g_v, rows_v, trans_v,
               gsem, wsem):
    wid = lax.axis_index("s") * NUM_CORES + lax.axis_index("c")
    b0 = wid * CHUNK
    pltpu.sync_copy(data_hbm.at[:, pl.ds(b0, CHUNK)], idx_v)

    lane = lax.iota(jnp.int32, 16)

    def issue_gather(j, slot):
        for bg in range(GROUPS):
            v = idx_v[j, pl.ds(bg * 16, 16)]
            g_v[slot, pl.ds(bg * 16, 16)] = lax.shift_right_logical(v, 2)
        pltpu.async_copy(tbl_hbm.at[g_v.at[slot]], rows_v.at[slot],
                         gsem.at[slot])

    for b in range(LOOKAHEAD):
        issue_gather(b, b)

    @pl.loop(0, NCH)
    def _(j):
        b = lax.rem(j, NBUF)
        pltpu.make_async_copy(
            tbl_hbm.at[g_v.at[b]], rows_v.at[b], gsem.at[b]).wait()

        jg = j + LOOKAHEAD
        bg_slot = lax.rem(jg, NBUF)

        @pl.when(jg < NCH)
        def _():
            @pl.when(jg >= NBUF)
            def _():
                jw = jg - NBUF
                pltpu.make_async_copy(
                    trans_v.at[bg_slot],
                    out_hbm.at[jw, :, pl.ds(b0, CHUNK)],
                    wsem.at[bg_slot]).wait()

            issue_gather(jg, bg_slot)

        qcol = [
            (idx_v[j, pl.ds(bg * 16, 16)] & 3) * 32 for bg in range(GROUPS)
        ]
        bvec = [lane + bg * 16 for bg in range(GROUPS)]

        @pl.loop(0, EMBED_DIM, unroll=4)
        def _(f):
            for bg in range(GROUPS):
                x = plsc.load_gather(rows_v.at[b], [bvec[bg], qcol[bg] + f])
                trans_v[b, f, pl.ds(bg * 16, 16)] = x

        pltpu.async_copy(
            trans_v.at[b], out_hbm.at[j, :, pl.ds(b0, CHUNK)], wsem.at[b])

    for t in range(NBUF):
        jw = NCH - NBUF + t
        b = jw % NBUF
        pltpu.make_async_copy(
            trans_v.at[b], out_hbm.at[jw, :, pl.ds(b0, CHUNK)],
            wsem.at[b]).wait()


def kernel(data, table):
    data_sm = jnp.transpose(data).astype(jnp.int32)
    tail = table[NSLAB * SLABW:].reshape(16, 128)
    tbl_g = _sc_regroup(jnp.transpose(table), tail)
    out_sm = _sc_gather(data_sm, tbl_g)
    return jnp.transpose(out_sm, (2, 0, 1))


# speedup vs baseline: 1.3167x; 1.3167x over previous
import functools

import jax
import jax.numpy as jnp
from jax import lax
from jax.experimental import pallas as pl
from jax.experimental.pallas import tpu as pltpu
from jax.experimental.pallas import tpu_sc as plsc

VOCAB = 1000000
EMBED_DIM = 32
BATCH = 4096
SEQ_LEN = 200

NUM_CORES = 2
NUM_SUBCORES = 16
NW = NUM_CORES * NUM_SUBCORES
CHUNK = 128
NCH = SEQ_LEN
NBUF = 5
LOOKAHEAD = 3
GROUPS = CHUNK // 16


_MESH = plsc.VectorSubcoreMesh(
    core_axis_name="c", subcore_axis_name="s",
    num_cores=NUM_CORES, num_subcores=NUM_SUBCORES,
)

SLABW = 512
SLABG = SLABW // 4
NSLAB = VOCAB // SLABW
KRING = 3


@functools.partial(
    pl.kernel,
    out_type=jax.ShapeDtypeStruct((VOCAB // 4, 128), jnp.float32),
    mesh=_MESH,
    scratch_types=[
        pltpu.VMEM((KRING, 32, SLABW), jnp.float32),
        pltpu.VMEM((KRING, SLABG, 128), jnp.float32),
        pltpu.SemaphoreType.DMA((KRING,)),
        pltpu.SemaphoreType.DMA((KRING,)),
    ],
    compiler_params=pltpu.CompilerParams(
        use_tc_tiling_on_sc=True, needs_layout_passes=False,
        disable_bounds_checks=True),
)
def _sc_regroup(tt_hbm, tail_hbm, out_hbm, slab_v, lin_v, lsem, ssem):
    wid = lax.axis_index("s") * NUM_CORES + lax.axis_index("c")
    nk = jnp.where(wid < NSLAB - (NSLAB // NW) * NW, NSLAB // NW + 1,
                   NSLAB // NW)

    @pl.when(wid == NW - 1)
    def _():
        pltpu.sync_copy(tail_hbm, out_hbm.at[pl.ds(NSLAB * SLABG, 16)])
    lane = lax.iota(jnp.int32, 16)
    fvec = [lane, lane + 16]

    def load(k, slot):
        v0 = (wid + NW * k) * SLABW
        pltpu.async_copy(tt_hbm.at[:, pl.ds(v0, SLABW)], slab_v.at[slot],
                         lsem.at[slot])

    @pl.when(nk > 0)
    def _():
        load(0, 0)

    @pl.when(nk > 1)
    def _():
        load(1, 1)

    @pl.loop(0, nk)
    def _(k):
        slot = lax.rem(k, KRING)
        v0 = (wid + NW * k) * SLABW
        pltpu.make_async_copy(
            tt_hbm.at[:, pl.ds(v0, SLABW)], slab_v.at[slot],
            lsem.at[slot]).wait()

        kn = k + 2
        sn = lax.rem(kn, KRING)

        @pl.when(kn < nk)
        def _():
            load(kn, sn)

        @pl.when(k >= KRING)
        def _():
            kp = k - KRING
            g0p = (wid + NW * kp) * SLABG
            pltpu.make_async_copy(
                lin_v.at[slot], out_hbm.at[pl.ds(g0p, SLABG)],
                ssem.at[slot]).wait()

        @pl.loop(0, SLABG, unroll=8)
        def _(r):
            col = [jnp.full((16,), 4 * r + a, jnp.int32) for a in range(4)]
            xs = [
                plsc.load_gather(slab_v.at[slot], [fvec[h % 2], col[h // 2]])
                for h in range(8)
            ]
            for h in range(8):
                lin_v[slot, r, pl.ds(16 * h, 16)] = xs[h]

        g0 = (wid + NW * k) * SLABG
        pltpu.async_copy(lin_v.at[slot], out_hbm.at[pl.ds(g0, SLABG)],
                         ssem.at[slot])

    @pl.loop(lax.max(nk - KRING, 0), nk)
    def _(k):
        slot = lax.rem(k, KRING)
        g0 = (wid + NW * k) * SLABG
        pltpu.make_async_copy(
            lin_v.at[slot], out_hbm.at[pl.ds(g0, SLABG)],
            ssem.at[slot]).wait()


@functools.partial(
    pl.kernel,
    out_type=jax.ShapeDtypeStruct((SEQ_LEN, EMBED_DIM, BATCH), jnp.float32),
    mesh=_MESH,
    scratch_types=[
        pltpu.VMEM((SEQ_LEN, CHUNK), jnp.int32),
        pltpu.VMEM((NBUF, CHUNK), jnp.int32),
        pltpu.VMEM((NBUF, CHUNK, 128), jnp.float32),
        pltpu.VMEM((NBUF, EMBED_DIM, CHUNK), jnp.float32),
        pltpu.SemaphoreType.DMA((NBUF,)),
        pltpu.SemaphoreType.DMA((NBUF,)),
    ],
    compiler_params=pltpu.CompilerParams(
        use_tc_tiling_on_sc=True, needs_layout_passes=False,
        disable_bounds_checks=True),
)
def _sc_gather(data_hbm, tbl_hbm, out_hbm, idx_v, g_v, rows_v, trans_v,
               gsem, wsem):
    wid = lax.axis_index("s") * NUM_CORES + lax.axis_index("c")
    b0 = wid * CHUNK
    pltpu.sync_copy(data_hbm.at[:, pl.ds(b0, CHUNK)], idx_v)

    lane = lax.iota(jnp.int32, 16)

    def issue_gather(j, slot):
        for bg in range(GROUPS):
            v = idx_v[j, pl.ds(bg * 16, 16)]
            g_v[slot, pl.ds(bg * 16, 16)] = lax.shift_right_logical(v, 2)
        pltpu.async_copy(tbl_hbm.at[g_v.at[slot]], rows_v.at[slot],
                         gsem.at[slot])

    for b in range(LOOKAHEAD):
        issue_gather(b, b)

    @pl.loop(0, NCH)
    def _(j):
        b = lax.rem(j, NBUF)
        pltpu.make_async_copy(
            tbl_hbm.at[g_v.at[b]], rows_v.at[b], gsem.at[b]).wait()

        jg = j + LOOKAHEAD
        bg_slot = lax.rem(jg, NBUF)

        @pl.when(jg < NCH)
        def _():
            @pl.when(jg >= NBUF)
            def _():
                jw = jg - NBUF
                pltpu.make_async_copy(
                    trans_v.at[bg_slot],
                    out_hbm.at[jw, :, pl.ds(b0, CHUNK)],
                    wsem.at[bg_slot]).wait()

            issue_gather(jg, bg_slot)

        qcol = [
            (idx_v[j, pl.ds(bg * 16, 16)] & 3) * 32 for bg in range(GROUPS)
        ]
        bvec = [lane + bg * 16 for bg in range(GROUPS)]

        @pl.loop(0, EMBED_DIM, unroll=8)
        def _(f):
            xs = [
                plsc.load_gather(rows_v.at[b], [bvec[bg], qcol[bg] + f])
                for bg in range(GROUPS)
            ]
            for bg in range(GROUPS):
                trans_v[b, f, pl.ds(bg * 16, 16)] = xs[bg]

        pltpu.async_copy(
            trans_v.at[b], out_hbm.at[j, :, pl.ds(b0, CHUNK)], wsem.at[b])

    for t in range(NBUF):
        jw = NCH - NBUF + t
        b = jw % NBUF
        pltpu.make_async_copy(
            trans_v.at[b], out_hbm.at[jw, :, pl.ds(b0, CHUNK)],
            wsem.at[b]).wait()


def kernel(data, table):
    data_sm = jnp.transpose(data).astype(jnp.int32)
    tail = table[NSLAB * SLABW:].reshape(16, 128)
    tbl_g = _sc_regroup(jnp.transpose(table), tail)
    out_sm = _sc_gather(data_sm, tbl_g)
    return jnp.transpose(out_sm, (2, 0, 1))


# speedup vs baseline: 1.3240x; 1.0056x over previous
import functools

import jax
import jax.numpy as jnp
from jax import lax
from jax.experimental import pallas as pl
from jax.experimental.pallas import tpu as pltpu
from jax.experimental.pallas import tpu_sc as plsc

VOCAB = 1000000
EMBED_DIM = 32
BATCH = 4096
SEQ_LEN = 200

NUM_CORES = 2
NUM_SUBCORES = 16
NW = NUM_CORES * NUM_SUBCORES
CHUNK = 128
NCH = SEQ_LEN
NBUF = 5
LOOKAHEAD = 3
GROUPS = CHUNK // 16


_MESH = plsc.VectorSubcoreMesh(
    core_axis_name="c", subcore_axis_name="s",
    num_cores=NUM_CORES, num_subcores=NUM_SUBCORES,
)

SLABW = 512
SLABG = SLABW // 4
NSLAB = VOCAB // SLABW
KRING = 3


@functools.partial(
    pl.kernel,
    out_type=jax.ShapeDtypeStruct((VOCAB // 4, 128), jnp.float32),
    mesh=_MESH,
    scratch_types=[
        pltpu.VMEM((KRING, 32, SLABW), jnp.float32),
        pltpu.VMEM((KRING, SLABG, 128), jnp.float32),
        pltpu.SemaphoreType.DMA((KRING,)),
        pltpu.SemaphoreType.DMA((KRING,)),
    ],
    compiler_params=pltpu.CompilerParams(
        use_tc_tiling_on_sc=True, needs_layout_passes=False,
        disable_bounds_checks=True),
)
def _sc_regroup(tt_hbm, tail_hbm, out_hbm, slab_v, lin_v, lsem, ssem):
    wid = lax.axis_index("s") * NUM_CORES + lax.axis_index("c")
    nk = jnp.where(wid < NSLAB - (NSLAB // NW) * NW, NSLAB // NW + 1,
                   NSLAB // NW)

    @pl.when(wid == NW - 1)
    def _():
        pltpu.sync_copy(tail_hbm, out_hbm.at[pl.ds(NSLAB * SLABG, 16)])
    lane = lax.iota(jnp.int32, 16)
    fvec = [lane, lane + 16]

    def load(k, slot):
        v0 = (wid + NW * k) * SLABW
        pltpu.async_copy(tt_hbm.at[:, pl.ds(v0, SLABW)], slab_v.at[slot],
                         lsem.at[slot])

    @pl.when(nk > 0)
    def _():
        load(0, 0)

    @pl.when(nk > 1)
    def _():
        load(1, 1)

    @pl.loop(0, nk)
    def _(k):
        slot = lax.rem(k, KRING)
        v0 = (wid + NW * k) * SLABW
        pltpu.make_async_copy(
            tt_hbm.at[:, pl.ds(v0, SLABW)], slab_v.at[slot],
            lsem.at[slot]).wait()

        kn = k + 2
        sn = lax.rem(kn, KRING)

        @pl.when(kn < nk)
        def _():
            load(kn, sn)

        @pl.when(k >= KRING)
        def _():
            kp = k - KRING
            g0p = (wid + NW * kp) * SLABG
            pltpu.make_async_copy(
                lin_v.at[slot], out_hbm.at[pl.ds(g0p, SLABG)],
                ssem.at[slot]).wait()

        @pl.loop(0, SLABG, unroll=16)
        def _(r):
            col = [jnp.full((16,), 4 * r + a, jnp.int32) for a in range(4)]
            xs = [
                plsc.load_gather(slab_v.at[slot], [fvec[h % 2], col[h // 2]])
                for h in range(8)
            ]
            for h in range(8):
                lin_v[slot, r, pl.ds(16 * h, 16)] = xs[h]

        g0 = (wid + NW * k) * SLABG
        pltpu.async_copy(lin_v.at[slot], out_hbm.at[pl.ds(g0, SLABG)],
                         ssem.at[slot])

    @pl.loop(lax.max(nk - KRING, 0), nk)
    def _(k):
        slot = lax.rem(k, KRING)
        g0 = (wid + NW * k) * SLABG
        pltpu.make_async_copy(
            lin_v.at[slot], out_hbm.at[pl.ds(g0, SLABG)],
            ssem.at[slot]).wait()


@functools.partial(
    pl.kernel,
    out_type=jax.ShapeDtypeStruct((SEQ_LEN, EMBED_DIM, BATCH), jnp.float32),
    mesh=_MESH,
    scratch_types=[
        pltpu.VMEM((SEQ_LEN, CHUNK), jnp.int32),
        pltpu.VMEM((NBUF, CHUNK), jnp.int32),
        pltpu.VMEM((NBUF, CHUNK, 128), jnp.float32),
        pltpu.VMEM((NBUF, EMBED_DIM, CHUNK), jnp.float32),
        pltpu.SemaphoreType.DMA((NBUF,)),
        pltpu.SemaphoreType.DMA((NBUF,)),
    ],
    compiler_params=pltpu.CompilerParams(
        use_tc_tiling_on_sc=True, needs_layout_passes=False,
        disable_bounds_checks=True),
)
def _sc_gather(data_hbm, tbl_hbm, out_hbm, idx_v, g_v, rows_v, trans_v,
               gsem, wsem):
    wid = lax.axis_index("s") * NUM_CORES + lax.axis_index("c")
    b0 = wid * CHUNK
    pltpu.sync_copy(data_hbm.at[:, pl.ds(b0, CHUNK)], idx_v)

    lane = lax.iota(jnp.int32, 16)

    def issue_gather(j, slot):
        for bg in range(GROUPS):
            v = idx_v[j, pl.ds(bg * 16, 16)]
            g_v[slot, pl.ds(bg * 16, 16)] = lax.shift_right_logical(v, 2)
        pltpu.async_copy(tbl_hbm.at[g_v.at[slot]], rows_v.at[slot],
                         gsem.at[slot])

    for b in range(LOOKAHEAD):
        issue_gather(b, b)

    @pl.loop(0, NCH)
    def _(j):
        b = lax.rem(j, NBUF)
        pltpu.make_async_copy(
            tbl_hbm.at[g_v.at[b]], rows_v.at[b], gsem.at[b]).wait()

        jg = j + LOOKAHEAD
        bg_slot = lax.rem(jg, NBUF)

        @pl.when(jg < NCH)
        def _():
            @pl.when(jg >= NBUF)
            def _():
                jw = jg - NBUF
                pltpu.make_async_copy(
                    trans_v.at[bg_slot],
                    out_hbm.at[jw, :, pl.ds(b0, CHUNK)],
                    wsem.at[bg_slot]).wait()

            issue_gather(jg, bg_slot)

        qcol = [
            (idx_v[j, pl.ds(bg * 16, 16)] & 3) * 32 for bg in range(GROUPS)
        ]
        bvec = [lane + bg * 16 for bg in range(GROUPS)]

        @pl.loop(0, EMBED_DIM, unroll=16)
        def _(f):
            xs = [
                plsc.load_gather(rows_v.at[b], [bvec[bg], qcol[bg] + f])
                for bg in range(GROUPS)
            ]
            for bg in range(GROUPS):
                trans_v[b, f, pl.ds(bg * 16, 16)] = xs[bg]

        pltpu.async_copy(
            trans_v.at[b], out_hbm.at[j, :, pl.ds(b0, CHUNK)], wsem.at[b])

    for t in range(NBUF):
        jw = NCH - NBUF + t
        b = jw % NBUF
        pltpu.make_async_copy(
            trans_v.at[b], out_hbm.at[jw, :, pl.ds(b0, CHUNK)],
            wsem.at[b]).wait()


def kernel(data, table):
    data_sm = jnp.transpose(data).astype(jnp.int32)
    tail = table[NSLAB * SLABW:].reshape(16, 128)
    tbl_g = _sc_regroup(jnp.transpose(table), tail)
    out_sm = _sc_gather(data_sm, tbl_g)
    return jnp.transpose(out_sm, (2, 0, 1))


# speedup vs baseline: 1.5001x; 1.1330x over previous
import functools

import jax
import jax.numpy as jnp
from jax import lax
from jax.experimental import pallas as pl
from jax.experimental.pallas import tpu as pltpu
from jax.experimental.pallas import tpu_sc as plsc

VOCAB = 1000000
EMBED_DIM = 32
BATCH = 4096
SEQ_LEN = 200

N_IDX = BATCH * SEQ_LEN
NUM_CORES = 2
NUM_SUBCORES = 16
NW = NUM_CORES * NUM_SUBCORES
CHUNK = 128
NCH = SEQ_LEN
NBUF = 8
LOOKAHEAD = 4

_MESH = plsc.VectorSubcoreMesh(
    core_axis_name="c", subcore_axis_name="s",
    num_cores=NUM_CORES, num_subcores=NUM_SUBCORES,
)


@functools.partial(
    pl.kernel,
    out_type=jax.ShapeDtypeStruct((SEQ_LEN, BATCH, EMBED_DIM), jnp.float32),
    mesh=_MESH,
    scratch_types=[
        pltpu.VMEM((NCH, CHUNK), jnp.int32),
        pltpu.VMEM((NBUF, CHUNK, EMBED_DIM), jnp.float32),
        pltpu.SemaphoreType.DMA((NBUF,)),
        pltpu.SemaphoreType.DMA((NBUF,)),
    ],
    compiler_params=pltpu.CompilerParams(use_tc_tiling_on_sc=False),
)
def _sc_gather(data_hbm, table_hbm, out_hbm, idx_v, rows_v, gsem, wsem):
    wid = lax.axis_index("s") * NUM_CORES + lax.axis_index("c")
    b0 = wid * CHUNK
    pltpu.sync_copy(data_hbm.at[:, wid], idx_v)

    for b in range(LOOKAHEAD):
        pltpu.async_copy(table_hbm.at[idx_v.at[b]], rows_v.at[b], gsem.at[b])

    @pl.loop(0, NCH)
    def _(j):
        b = lax.rem(j, NBUF)
        pltpu.make_async_copy(
            table_hbm.at[idx_v.at[j]], rows_v.at[b], gsem.at[b]).wait()
        pltpu.async_copy(
            rows_v.at[b], out_hbm.at[j, pl.ds(b0, CHUNK)], wsem.at[b])

        jg = j + LOOKAHEAD
        bg = lax.rem(jg, NBUF)

        @pl.when(jg < NCH)
        def _():
            @pl.when(jg >= NBUF)
            def _():
                jw = jg - NBUF
                pltpu.make_async_copy(
                    rows_v.at[bg],
                    out_hbm.at[jw, pl.ds(b0, CHUNK)],
                    wsem.at[bg]).wait()

            pltpu.async_copy(
                table_hbm.at[idx_v.at[jg]], rows_v.at[bg], gsem.at[bg])

    for t in range(NBUF):
        jw = NCH - NBUF + t
        b = jw % NBUF
        pltpu.make_async_copy(
            rows_v.at[b], out_hbm.at[jw, pl.ds(b0, CHUNK)],
            wsem.at[b]).wait()


def kernel(data, table):
    data_sm = jnp.transpose(data).reshape(SEQ_LEN, NW, CHUNK).astype(jnp.int32)
    out_sm = _sc_gather(data_sm, table)
    return jnp.transpose(out_sm, (1, 0, 2))
